# baseline (speedup 1.0000x reference)
import functools

import jax
import jax.numpy as jnp
from jax.experimental import pallas as pl
from jax.experimental.pallas import tpu as pltpu

N_NEURONS = 262144
D_MODEL = 768
TOP_K = 8
N_QUERIES = 8
BLOCK_N = 2048

_NEG_INF = float("-inf")
_BIG_I32 = 2**30


def _insert_sorted(rs, ri, m, j):
    lt = rs < m
    ss = jnp.concatenate([m, rs[:, :-1]], axis=1)
    si = jnp.concatenate([j, ri[:, :-1]], axis=1)
    ltp = ss < m
    new_s = jnp.where(lt, jnp.where(ltp, ss, m), rs)
    new_i = jnp.where(lt, jnp.where(ltp, si, j), ri)
    return new_s, new_i


def _topk_body(x_ref, wqt_ref, bq_ref, n_ref, idx_out, w_out,
               q_s, run_s, run_i, s_buf):
    i = pl.program_id(0)
    nb = pl.num_programs(0)

    @pl.when(i == 0)
    def _init():
        q_s[...] = (
            jnp.dot(x_ref[...], wqt_ref[...], preferred_element_type=jnp.float32)
            + bq_ref[...]
        )
        run_s[...] = jnp.full((N_QUERIES, TOP_K), _NEG_INF, jnp.float32)
        run_i[...] = jnp.full((N_QUERIES, TOP_K), -1, jnp.int32)

    s = jax.lax.dot_general(
        q_s[...], n_ref[...], (((1,), (1,)), ((), ())),
        preferred_element_type=jnp.float32)
    s_buf[...] = s
    col = (jax.lax.broadcasted_iota(jnp.int32, (N_QUERIES, BLOCK_N), 1)
           + i * BLOCK_N)

    thr = run_s[...][:, TOP_K - 1:TOP_K]
    cnt = jnp.sum((s > thr).astype(jnp.int32), axis=1)
    t = jnp.max(cnt)

    for k in range(TOP_K):
        @pl.when(t > k)
        def _step():
            sb = s_buf[...]
            m = jnp.max(sb, axis=1, keepdims=True)
            cand = jnp.where(sb == m, col, _BIG_I32)
            j = jnp.min(cand, axis=1, keepdims=True)
            s_buf[...] = jnp.where(col == j, _NEG_INF, sb)
            ns, ni = _insert_sorted(run_s[...], run_i[...], m, j)
            run_s[...] = ns
            run_i[...] = ni

    @pl.when(i == nb - 1)
    def _fin():
        fs = run_s[...]
        e = jnp.exp(fs - fs[:, :1])
        w_out[...] = e / jnp.sum(e, axis=1, keepdims=True)
        idx_out[...] = run_i[...]


def _topk_call(x2d, neurons, wqt, bq2d):
    nb = N_NEURONS // BLOCK_N
    return pl.pallas_call(
        _topk_body,
        grid=(nb,),
        in_specs=[
            pl.BlockSpec((N_QUERIES, D_MODEL), lambda i: (0, 0)),
            pl.BlockSpec((D_MODEL, D_MODEL), lambda i: (0, 0)),
            pl.BlockSpec((1, D_MODEL), lambda i: (0, 0)),
            pl.BlockSpec((BLOCK_N, D_MODEL), lambda i: (i, 0)),
        ],
        out_specs=[
            pl.BlockSpec((N_QUERIES, TOP_K), lambda i: (0, 0)),
            pl.BlockSpec((N_QUERIES, TOP_K), lambda i: (0, 0)),
        ],
        out_shape=[
            jax.ShapeDtypeStruct((N_QUERIES, TOP_K), jnp.int32),
            jax.ShapeDtypeStruct((N_QUERIES, TOP_K), jnp.float32),
        ],
        scratch_shapes=[
            pltpu.VMEM((N_QUERIES, D_MODEL), jnp.float32),
            pltpu.VMEM((N_QUERIES, TOP_K), jnp.float32),
            pltpu.VMEM((N_QUERIES, TOP_K), jnp.int32),
            pltpu.VMEM((N_QUERIES, BLOCK_N), jnp.float32),
        ],
        compiler_params=pltpu.CompilerParams(
            dimension_semantics=("arbitrary",)),
    )(x2d, wqt, bq2d, neurons)


def _gather_body(idx_ref, w_ref, row_ref, out_ref):
    i = pl.program_id(0)
    k = i % TOP_K

    @pl.when(k == 0)
    def _z():
        out_ref[...] = jnp.zeros_like(out_ref)

    out_ref[...] += w_ref[i] * row_ref[...]


def _gather_call(idx_flat, w_flat, neurons):
    grid_spec = pltpu.PrefetchScalarGridSpec(
        num_scalar_prefetch=2,
        grid=(N_QUERIES * TOP_K,),
        in_specs=[
            pl.BlockSpec((1, 1, D_MODEL), lambda i, idx, w: (idx[i], 0, 0)),
        ],
        out_specs=pl.BlockSpec(
            (1, 1, D_MODEL), lambda i, idx, w: (i // TOP_K, 0, 0)),
    )
    return pl.pallas_call(
        _gather_body,
        grid_spec=grid_spec,
        out_shape=jax.ShapeDtypeStruct((N_QUERIES, 1, D_MODEL), jnp.float32),
        compiler_params=pltpu.CompilerParams(
            dimension_semantics=("arbitrary",)),
    )(idx_flat, w_flat, neurons.reshape(N_NEURONS, 1, D_MODEL))


@jax.jit
def kernel(x, neurons, W_q, b_q):
    x2d = x.reshape(N_QUERIES, D_MODEL)
    wqt = W_q.T
    bq2d = b_q.reshape(1, D_MODEL)
    topk_idx, topk_w = _topk_call(x2d, neurons, wqt, bq2d)
    out = _gather_call(topk_idx.reshape(-1), topk_w.reshape(-1), neurons)
    return (
        out,
        topk_idx.reshape(N_QUERIES, 1, TOP_K),
        topk_w.reshape(N_QUERIES, 1, TOP_K),
    )


# speedup vs baseline: 2.8710x; 2.8710x over previous
import functools

import jax
import jax.numpy as jnp
from jax import lax
from jax.experimental import pallas as pl
from jax.experimental.pallas import tpu as pltpu
from jax.experimental.pallas import tpu_sc as plsc

N_NEURONS = 262144
D_MODEL = 768
TOP_K = 8
N_QUERIES = 8
BLOCK_N = 2048

_NEG_INF = float("-inf")
_BIG_I32 = 2**30


def _insert_sorted(rs, ri, m, j):
    lt = rs < m
    ss = jnp.concatenate([m, rs[:, :-1]], axis=1)
    si = jnp.concatenate([j, ri[:, :-1]], axis=1)
    ltp = ss < m
    new_s = jnp.where(lt, jnp.where(ltp, ss, m), rs)
    new_i = jnp.where(lt, jnp.where(ltp, si, j), ri)
    return new_s, new_i


def _topk_body(x_ref, wqt_ref, bq_ref, n_ref, idx_out, w_out,
               q_s, run_s, run_i, s_buf):
    i = pl.program_id(0)
    nb = pl.num_programs(0)

    @pl.when(i == 0)
    def _init():
        q_s[...] = (
            jnp.dot(x_ref[...], wqt_ref[...], preferred_element_type=jnp.float32)
            + bq_ref[...]
        )
        run_s[...] = jnp.full((N_QUERIES, TOP_K), _NEG_INF, jnp.float32)
        run_i[...] = jnp.full((N_QUERIES, TOP_K), -1, jnp.int32)

    s = jax.lax.dot_general(
        q_s[...], n_ref[...], (((1,), (1,)), ((), ())),
        preferred_element_type=jnp.float32)
    s_buf[...] = s
    col = (jax.lax.broadcasted_iota(jnp.int32, (N_QUERIES, BLOCK_N), 1)
           + i * BLOCK_N)

    thr = run_s[...][:, TOP_K - 1:TOP_K]
    cnt = jnp.sum((s > thr).astype(jnp.int32), axis=1)
    t = jnp.max(cnt)

    for k in range(TOP_K):
        @pl.when(t > k)
        def _step():
            sb = s_buf[...]
            m = jnp.max(sb, axis=1, keepdims=True)
            cand = jnp.where(sb == m, col, _BIG_I32)
            j = jnp.min(cand, axis=1, keepdims=True)
            s_buf[...] = jnp.where(col == j, _NEG_INF, sb)
            ns, ni = _insert_sorted(run_s[...], run_i[...], m, j)
            run_s[...] = ns
            run_i[...] = ni

    @pl.when(i == nb - 1)
    def _fin():
        fs = run_s[...]
        e = jnp.exp(fs - fs[:, :1])
        w_out[...] = e / jnp.sum(e, axis=1, keepdims=True)
        idx_out[...] = run_i[...]


def _topk_call(x2d, neurons, wqt, bq2d):
    nb = N_NEURONS // BLOCK_N
    return pl.pallas_call(
        _topk_body,
        grid=(nb,),
        in_specs=[
            pl.BlockSpec((N_QUERIES, D_MODEL), lambda i: (0, 0)),
            pl.BlockSpec((D_MODEL, D_MODEL), lambda i: (0, 0)),
            pl.BlockSpec((1, D_MODEL), lambda i: (0, 0)),
            pl.BlockSpec((BLOCK_N, D_MODEL), lambda i: (i, 0)),
        ],
        out_specs=[
            pl.BlockSpec((N_QUERIES, TOP_K), lambda i: (0, 0)),
            pl.BlockSpec((N_QUERIES, TOP_K), lambda i: (0, 0)),
        ],
        out_shape=[
            jax.ShapeDtypeStruct((N_QUERIES, TOP_K), jnp.int32),
            jax.ShapeDtypeStruct((N_QUERIES, TOP_K), jnp.float32),
        ],
        scratch_shapes=[
            pltpu.VMEM((N_QUERIES, D_MODEL), jnp.float32),
            pltpu.VMEM((N_QUERIES, TOP_K), jnp.float32),
            pltpu.VMEM((N_QUERIES, TOP_K), jnp.int32),
            pltpu.VMEM((N_QUERIES, BLOCK_N), jnp.float32),
        ],
        compiler_params=pltpu.CompilerParams(
            dimension_semantics=("arbitrary",)),
    )(x2d, wqt, bq2d, neurons)


_LANES = 16


def _sc_gather_body(idx_hbm, wb_hbm, table_hbm, out_hbm,
                    idx_v, wb_v, rows_v, acc_v, sem):
    wid = lax.axis_index("s") * 2 + lax.axis_index("c")

    @pl.when(wid < N_QUERIES)
    def _():
        b = wid
        pltpu.sync_copy(idx_hbm, idx_v)
        pltpu.sync_copy(wb_hbm, wb_v)
        pltpu.async_copy(table_hbm.at[idx_v], rows_v, sem).wait()
        wk = [wb_v[pl.ds((b * TOP_K + k) * _LANES, _LANES)]
              for k in range(TOP_K)]
        for j in range(D_MODEL // _LANES):
            a = wk[0] * rows_v[b * TOP_K + 0, pl.ds(j * _LANES, _LANES)]
            for k in range(1, TOP_K):
                a = a + wk[k] * rows_v[b * TOP_K + k, pl.ds(j * _LANES, _LANES)]
            acc_v[pl.ds(j * _LANES, _LANES)] = a
        pltpu.sync_copy(acc_v, out_hbm.at[b])


def _gather_call(idx_flat, w_flat, neurons):
    w_bcast = jnp.broadcast_to(
        w_flat[:, None], (N_QUERIES * TOP_K, _LANES)).reshape(-1)
    mesh = plsc.VectorSubcoreMesh(core_axis_name="c", subcore_axis_name="s")
    f = pl.kernel(
        _sc_gather_body, mesh=mesh,
        out_type=jax.ShapeDtypeStruct((N_QUERIES, D_MODEL), jnp.float32),
        scratch_types=[
            pltpu.VMEM((N_QUERIES * TOP_K,), jnp.int32),
            pltpu.VMEM((N_QUERIES * TOP_K * _LANES,), jnp.float32),
            pltpu.VMEM((N_QUERIES * TOP_K, D_MODEL), jnp.float32),
            pltpu.VMEM((D_MODEL,), jnp.float32),
            pltpu.SemaphoreType.DMA,
        ],
    )
    return f(idx_flat, w_bcast, neurons)


@jax.jit
def kernel(x, neurons, W_q, b_q):
    x2d = x.reshape(N_QUERIES, D_MODEL)
    wqt = W_q.T
    bq2d = b_q.reshape(1, D_MODEL)
    topk_idx, topk_w = _topk_call(x2d, neurons, wqt, bq2d)
    out = _gather_call(topk_idx.reshape(-1), topk_w.reshape(-1), neurons)
    return (
        out.reshape(N_QUERIES, 1, D_MODEL),
        topk_idx.reshape(N_QUERIES, 1, TOP_K),
        topk_w.reshape(N_QUERIES, 1, TOP_K),
    )


# speedup vs baseline: 3.3506x; 1.1671x over previous
import functools

import jax
import jax.numpy as jnp
from jax import lax
from jax.experimental import pallas as pl
from jax.experimental.pallas import tpu as pltpu
from jax.experimental.pallas import tpu_sc as plsc

N_NEURONS = 262144
D_MODEL = 768
TOP_K = 8
N_QUERIES = 8
BLOCK_N = 4096

_NEG_INF = float("-inf")
_BIG_I32 = 2**30


def _insert_sorted(rs, ri, m, j):
    lt = rs < m
    ss = jnp.concatenate([m, rs[:, :-1]], axis=1)
    si = jnp.concatenate([j, ri[:, :-1]], axis=1)
    ltp = ss < m
    new_s = jnp.where(lt, jnp.where(ltp, ss, m), rs)
    new_i = jnp.where(lt, jnp.where(ltp, si, j), ri)
    return new_s, new_i


def _topk_body(x_ref, wqt_ref, bq_ref, n_ref, idx_out, w_out,
               q_s, run_s, run_i, s_buf):
    i = pl.program_id(0)
    nb = pl.num_programs(0)

    @pl.when(i == 0)
    def _init():
        q_s[...] = (
            jnp.dot(x_ref[...], wqt_ref[...], preferred_element_type=jnp.float32)
            + bq_ref[...]
        )
        run_s[...] = jnp.full((N_QUERIES, TOP_K), _NEG_INF, jnp.float32)
        run_i[...] = jnp.full((N_QUERIES, TOP_K), -1, jnp.int32)

    s = jax.lax.dot_general(
        q_s[...], n_ref[...], (((1,), (1,)), ((), ())),
        preferred_element_type=jnp.float32)
    s_buf[...] = s
    col = (jax.lax.broadcasted_iota(jnp.int32, (N_QUERIES, BLOCK_N), 1)
           + i * BLOCK_N)

    thr = run_s[...][:, TOP_K - 1:TOP_K]
    cnt = jnp.sum((s > thr).astype(jnp.int32), axis=1)
    t = jnp.max(cnt)

    for k in range(TOP_K):
        @pl.when(t > k)
        def _step():
            sb = s_buf[...]
            m = jnp.max(sb, axis=1, keepdims=True)
            cand = jnp.where(sb == m, col, _BIG_I32)
            j = jnp.min(cand, axis=1, keepdims=True)
            s_buf[...] = jnp.where(col == j, _NEG_INF, sb)
            ns, ni = _insert_sorted(run_s[...], run_i[...], m, j)
            run_s[...] = ns
            run_i[...] = ni

    @pl.when(i == nb - 1)
    def _fin():
        fs = run_s[...]
        e = jnp.exp(fs - fs[:, :1])
        w_out[...] = e / jnp.sum(e, axis=1, keepdims=True)
        idx_out[...] = run_i[...]


def _topk_call(x2d, neurons, wqt, bq2d):
    nb = N_NEURONS // BLOCK_N
    return pl.pallas_call(
        _topk_body,
        grid=(nb,),
        in_specs=[
            pl.BlockSpec((N_QUERIES, D_MODEL), lambda i: (0, 0)),
            pl.BlockSpec((D_MODEL, D_MODEL), lambda i: (0, 0)),
            pl.BlockSpec((1, D_MODEL), lambda i: (0, 0)),
            pl.BlockSpec((BLOCK_N, D_MODEL), lambda i: (i, 0)),
        ],
        out_specs=[
            pl.BlockSpec((N_QUERIES, TOP_K), lambda i: (0, 0)),
            pl.BlockSpec((N_QUERIES, TOP_K), lambda i: (0, 0)),
        ],
        out_shape=[
            jax.ShapeDtypeStruct((N_QUERIES, TOP_K), jnp.int32),
            jax.ShapeDtypeStruct((N_QUERIES, TOP_K), jnp.float32),
        ],
        scratch_shapes=[
            pltpu.VMEM((N_QUERIES, D_MODEL), jnp.float32),
            pltpu.VMEM((N_QUERIES, TOP_K), jnp.float32),
            pltpu.VMEM((N_QUERIES, TOP_K), jnp.int32),
            pltpu.VMEM((N_QUERIES, BLOCK_N), jnp.float32),
        ],
        compiler_params=pltpu.CompilerParams(
            dimension_semantics=("arbitrary",)),
    )(x2d, wqt, bq2d, neurons)


_LANES = 16


def _sc_gather_body(idx_hbm, wb_hbm, table_hbm, out_hbm,
                    idx_v, wb_v, rows_v, acc_v, sem):
    wid = lax.axis_index("s") * 2 + lax.axis_index("c")

    @pl.when(wid < N_QUERIES)
    def _():
        b = wid
        pltpu.sync_copy(idx_hbm, idx_v)
        pltpu.sync_copy(wb_hbm, wb_v)
        pltpu.async_copy(table_hbm.at[idx_v], rows_v, sem).wait()
        wk = [wb_v[pl.ds((b * TOP_K + k) * _LANES, _LANES)]
              for k in range(TOP_K)]
        for j in range(D_MODEL // _LANES):
            a = wk[0] * rows_v[b * TOP_K + 0, pl.ds(j * _LANES, _LANES)]
            for k in range(1, TOP_K):
                a = a + wk[k] * rows_v[b * TOP_K + k, pl.ds(j * _LANES, _LANES)]
            acc_v[pl.ds(j * _LANES, _LANES)] = a
        pltpu.sync_copy(acc_v, out_hbm.at[b])


def _gather_call(idx_flat, w_flat, neurons):
    w_bcast = jnp.broadcast_to(
        w_flat[:, None], (N_QUERIES * TOP_K, _LANES)).reshape(-1)
    mesh = plsc.VectorSubcoreMesh(core_axis_name="c", subcore_axis_name="s")
    f = pl.kernel(
        _sc_gather_body, mesh=mesh,
        out_type=jax.ShapeDtypeStruct((N_QUERIES, D_MODEL), jnp.float32),
        scratch_types=[
            pltpu.VMEM((N_QUERIES * TOP_K,), jnp.int32),
            pltpu.VMEM((N_QUERIES * TOP_K * _LANES,), jnp.float32),
            pltpu.VMEM((N_QUERIES * TOP_K, D_MODEL), jnp.float32),
            pltpu.VMEM((D_MODEL,), jnp.float32),
            pltpu.SemaphoreType.DMA,
        ],
    )
    return f(idx_flat, w_bcast, neurons)


@jax.jit
def kernel(x, neurons, W_q, b_q):
    x2d = x.reshape(N_QUERIES, D_MODEL)
    wqt = W_q.T
    bq2d = b_q.reshape(1, D_MODEL)
    topk_idx, topk_w = _topk_call(x2d, neurons, wqt, bq2d)
    out = _gather_call(topk_idx.reshape(-1), topk_w.reshape(-1), neurons)
    return (
        out.reshape(N_QUERIES, 1, D_MODEL),
        topk_idx.reshape(N_QUERIES, 1, TOP_K),
        topk_w.reshape(N_QUERIES, 1, TOP_K),
    )


# speedup vs baseline: 3.4131x; 1.0187x over previous
import functools

import jax
import jax.numpy as jnp
from jax import lax
from jax.experimental import pallas as pl
from jax.experimental.pallas import tpu as pltpu
from jax.experimental.pallas import tpu_sc as plsc

N_NEURONS = 262144
D_MODEL = 768
TOP_K = 8
N_QUERIES = 8
BLOCK_N = 8192

_NEG_INF = float("-inf")
_BIG_I32 = 2**30


def _insert_sorted(rs, ri, m, j):
    lt = rs < m
    ss = jnp.concatenate([m, rs[:, :-1]], axis=1)
    si = jnp.concatenate([j, ri[:, :-1]], axis=1)
    ltp = ss < m
    new_s = jnp.where(lt, jnp.where(ltp, ss, m), rs)
    new_i = jnp.where(lt, jnp.where(ltp, si, j), ri)
    return new_s, new_i


def _topk_body(x_ref, wqt_ref, bq_ref, n_ref, idx_out, w_out,
               q_s, run_s, run_i, s_buf):
    i = pl.program_id(0)
    nb = pl.num_programs(0)

    @pl.when(i == 0)
    def _init():
        q_s[...] = (
            jnp.dot(x_ref[...], wqt_ref[...], preferred_element_type=jnp.float32)
            + bq_ref[...]
        )
        run_s[...] = jnp.full((N_QUERIES, TOP_K), _NEG_INF, jnp.float32)
        run_i[...] = jnp.full((N_QUERIES, TOP_K), -1, jnp.int32)

    s = jax.lax.dot_general(
        q_s[...], n_ref[...], (((1,), (1,)), ((), ())),
        preferred_element_type=jnp.float32)
    s_buf[...] = s
    col = (jax.lax.broadcasted_iota(jnp.int32, (N_QUERIES, BLOCK_N), 1)
           + i * BLOCK_N)

    thr = run_s[...][:, TOP_K - 1:TOP_K]
    cnt = jnp.sum((s > thr).astype(jnp.int32), axis=1)
    t = jnp.max(cnt)

    for k in range(TOP_K):
        @pl.when(t > k)
        def _step():
            sb = s_buf[...]
            m = jnp.max(sb, axis=1, keepdims=True)
            cand = jnp.where(sb == m, col, _BIG_I32)
            j = jnp.min(cand, axis=1, keepdims=True)
            s_buf[...] = jnp.where(col == j, _NEG_INF, sb)
            ns, ni = _insert_sorted(run_s[...], run_i[...], m, j)
            run_s[...] = ns
            run_i[...] = ni

    @pl.when(i == nb - 1)
    def _fin():
        fs = run_s[...]
        e = jnp.exp(fs - fs[:, :1])
        w_out[...] = e / jnp.sum(e, axis=1, keepdims=True)
        idx_out[...] = run_i[...]


def _topk_call(x2d, neurons, wqt, bq2d):
    nb = N_NEURONS // BLOCK_N
    return pl.pallas_call(
        _topk_body,
        grid=(nb,),
        in_specs=[
            pl.BlockSpec((N_QUERIES, D_MODEL), lambda i: (0, 0)),
            pl.BlockSpec((D_MODEL, D_MODEL), lambda i: (0, 0)),
            pl.BlockSpec((1, D_MODEL), lambda i: (0, 0)),
            pl.BlockSpec((BLOCK_N, D_MODEL), lambda i: (i, 0)),
        ],
        out_specs=[
            pl.BlockSpec((N_QUERIES, TOP_K), lambda i: (0, 0)),
            pl.BlockSpec((N_QUERIES, TOP_K), lambda i: (0, 0)),
        ],
        out_shape=[
            jax.ShapeDtypeStruct((N_QUERIES, TOP_K), jnp.int32),
            jax.ShapeDtypeStruct((N_QUERIES, TOP_K), jnp.float32),
        ],
        scratch_shapes=[
            pltpu.VMEM((N_QUERIES, D_MODEL), jnp.float32),
            pltpu.VMEM((N_QUERIES, TOP_K), jnp.float32),
            pltpu.VMEM((N_QUERIES, TOP_K), jnp.int32),
            pltpu.VMEM((N_QUERIES, BLOCK_N), jnp.float32),
        ],
        compiler_params=pltpu.CompilerParams(
            dimension_semantics=("arbitrary",)),
    )(x2d, wqt, bq2d, neurons)


_LANES = 16


def _sc_gather_body(idx_hbm, wb_hbm, table_hbm, out_hbm,
                    idx_v, wb_v, rows_v, acc_v, sem):
    wid = lax.axis_index("s") * 2 + lax.axis_index("c")

    @pl.when(wid < N_QUERIES)
    def _():
        b = wid
        pltpu.sync_copy(idx_hbm, idx_v)
        pltpu.sync_copy(wb_hbm, wb_v)
        pltpu.async_copy(table_hbm.at[idx_v], rows_v, sem).wait()
        wk = [wb_v[pl.ds((b * TOP_K + k) * _LANES, _LANES)]
              for k in range(TOP_K)]
        for j in range(D_MODEL // _LANES):
            a = wk[0] * rows_v[b * TOP_K + 0, pl.ds(j * _LANES, _LANES)]
            for k in range(1, TOP_K):
                a = a + wk[k] * rows_v[b * TOP_K + k, pl.ds(j * _LANES, _LANES)]
            acc_v[pl.ds(j * _LANES, _LANES)] = a
        pltpu.sync_copy(acc_v, out_hbm.at[b])


def _gather_call(idx_flat, w_flat, neurons):
    w_bcast = jnp.broadcast_to(
        w_flat[:, None], (N_QUERIES * TOP_K, _LANES)).reshape(-1)
    mesh = plsc.VectorSubcoreMesh(core_axis_name="c", subcore_axis_name="s")
    f = pl.kernel(
        _sc_gather_body, mesh=mesh,
        out_type=jax.ShapeDtypeStruct((N_QUERIES, D_MODEL), jnp.float32),
        scratch_types=[
            pltpu.VMEM((N_QUERIES * TOP_K,), jnp.int32),
            pltpu.VMEM((N_QUERIES * TOP_K * _LANES,), jnp.float32),
            pltpu.VMEM((N_QUERIES * TOP_K, D_MODEL), jnp.float32),
            pltpu.VMEM((D_MODEL,), jnp.float32),
            pltpu.SemaphoreType.DMA,
        ],
    )
    return f(idx_flat, w_bcast, neurons)


@jax.jit
def kernel(x, neurons, W_q, b_q):
    x2d = x.reshape(N_QUERIES, D_MODEL)
    wqt = W_q.T
    bq2d = b_q.reshape(1, D_MODEL)
    topk_idx, topk_w = _topk_call(x2d, neurons, wqt, bq2d)
    out = _gather_call(topk_idx.reshape(-1), topk_w.reshape(-1), neurons)
    return (
        out.reshape(N_QUERIES, 1, D_MODEL),
        topk_idx.reshape(N_QUERIES, 1, TOP_K),
        topk_w.reshape(N_QUERIES, 1, TOP_K),
    )


# speedup vs baseline: 3.4600x; 1.0137x over previous
import functools

import jax
import jax.numpy as jnp
from jax import lax
from jax.experimental import pallas as pl
from jax.experimental.pallas import tpu as pltpu
from jax.experimental.pallas import tpu_sc as plsc

N_NEURONS = 262144
D_MODEL = 768
TOP_K = 8
N_QUERIES = 8
BLOCK_N = 8192

_NEG_INF = float("-inf")
_BIG_I32 = 2**30


def _insert_sorted(rs, ri, m, j):
    lt = rs < m
    ss = jnp.concatenate([m, rs[:, :-1]], axis=1)
    si = jnp.concatenate([j, ri[:, :-1]], axis=1)
    ltp = ss < m
    new_s = jnp.where(lt, jnp.where(ltp, ss, m), rs)
    new_i = jnp.where(lt, jnp.where(ltp, si, j), ri)
    return new_s, new_i


def _topk_body(x_ref, wqt_ref, bq_ref, n_ref, idx_out, w_out,
               q_s, run_s, run_i, s_buf):
    i = pl.program_id(0)
    nb = pl.num_programs(0)

    @pl.when(i == 0)
    def _init():
        q_s[...] = (
            jnp.dot(x_ref[...], wqt_ref[...], preferred_element_type=jnp.float32)
            + bq_ref[...]
        )
        run_s[...] = jnp.full((N_QUERIES, TOP_K), _NEG_INF, jnp.float32)
        run_i[...] = jnp.full((N_QUERIES, TOP_K), -1, jnp.int32)

    s = jax.lax.dot_general(
        q_s[...], n_ref[...], (((1,), (1,)), ((), ())),
        preferred_element_type=jnp.float32)
    s_buf[...] = s
    col = (jax.lax.broadcasted_iota(jnp.int32, (N_QUERIES, BLOCK_N), 1)
           + i * BLOCK_N)

    thr = run_s[...][:, TOP_K - 1:TOP_K]
    cnt = jnp.sum((s > thr).astype(jnp.int32), axis=1)
    t = jnp.max(cnt)

    for k in range(TOP_K):
        @pl.when(t > k)
        def _step():
            sb = s_buf[...]
            m = jnp.max(sb, axis=1, keepdims=True)
            cand = jnp.where(sb == m, col, _BIG_I32)
            j = jnp.min(cand, axis=1, keepdims=True)
            s_buf[...] = jnp.where(col == j, _NEG_INF, sb)
            ns, ni = _insert_sorted(run_s[...], run_i[...], m, j)
            run_s[...] = ns
            run_i[...] = ni

    @pl.when(i == nb - 1)
    def _fin():
        fs = run_s[...]
        e = jnp.exp(fs - fs[:, :1])
        w_out[...] = e / jnp.sum(e, axis=1, keepdims=True)
        idx_out[...] = run_i[...]


def _topk_call(x2d, neurons, wqt, bq2d):
    nb = N_NEURONS // BLOCK_N
    return pl.pallas_call(
        _topk_body,
        grid=(nb,),
        in_specs=[
            pl.BlockSpec((N_QUERIES, D_MODEL), lambda i: (0, 0)),
            pl.BlockSpec((D_MODEL, D_MODEL), lambda i: (0, 0)),
            pl.BlockSpec((1, D_MODEL), lambda i: (0, 0)),
            pl.BlockSpec((BLOCK_N, D_MODEL), lambda i: (i, 0)),
        ],
        out_specs=[
            pl.BlockSpec((N_QUERIES, TOP_K), lambda i: (0, 0)),
            pl.BlockSpec((N_QUERIES, TOP_K), lambda i: (0, 0)),
        ],
        out_shape=[
            jax.ShapeDtypeStruct((N_QUERIES, TOP_K), jnp.int32),
            jax.ShapeDtypeStruct((N_QUERIES, TOP_K), jnp.float32),
        ],
        scratch_shapes=[
            pltpu.VMEM((N_QUERIES, D_MODEL), jnp.float32),
            pltpu.VMEM((N_QUERIES, TOP_K), jnp.float32),
            pltpu.VMEM((N_QUERIES, TOP_K), jnp.int32),
            pltpu.VMEM((N_QUERIES, BLOCK_N), jnp.float32),
        ],
        compiler_params=pltpu.CompilerParams(
            dimension_semantics=("arbitrary",)),
    )(x2d, wqt, bq2d, neurons)


_LANES = 16


def _sc_gather_body(idx_hbm, wb_hbm, table_hbm, out_hbm,
                    idx_v, wb_v, rows_v, acc_v, sem):
    wid = lax.axis_index("s") * 2 + lax.axis_index("c")

    @pl.when(wid < N_QUERIES)
    def _():
        b = wid
        pltpu.sync_copy(idx_hbm.at[pl.ds(b * TOP_K, TOP_K)], idx_v)
        pltpu.sync_copy(
            wb_hbm.at[pl.ds(b * TOP_K * _LANES, TOP_K * _LANES)], wb_v)
        pltpu.async_copy(table_hbm.at[idx_v], rows_v, sem).wait()
        wk = [wb_v[pl.ds(k * _LANES, _LANES)] for k in range(TOP_K)]
        for j in range(D_MODEL // _LANES):
            a = wk[0] * rows_v[0, pl.ds(j * _LANES, _LANES)]
            for k in range(1, TOP_K):
                a = a + wk[k] * rows_v[k, pl.ds(j * _LANES, _LANES)]
            acc_v[pl.ds(j * _LANES, _LANES)] = a
        pltpu.sync_copy(acc_v, out_hbm.at[b])


def _gather_call(idx_flat, w_flat, neurons):
    w_bcast = jnp.broadcast_to(
        w_flat[:, None], (N_QUERIES * TOP_K, _LANES)).reshape(-1)
    mesh = plsc.VectorSubcoreMesh(core_axis_name="c", subcore_axis_name="s")
    f = pl.kernel(
        _sc_gather_body, mesh=mesh,
        out_type=jax.ShapeDtypeStruct((N_QUERIES, D_MODEL), jnp.float32),
        scratch_types=[
            pltpu.VMEM((TOP_K,), jnp.int32),
            pltpu.VMEM((TOP_K * _LANES,), jnp.float32),
            pltpu.VMEM((TOP_K, D_MODEL), jnp.float32),
            pltpu.VMEM((D_MODEL,), jnp.float32),
            pltpu.SemaphoreType.DMA,
        ],
    )
    return f(idx_flat, w_bcast, neurons)


@jax.jit
def kernel(x, neurons, W_q, b_q):
    x2d = x.reshape(N_QUERIES, D_MODEL)
    wqt = W_q.T
    bq2d = b_q.reshape(1, D_MODEL)
    topk_idx, topk_w = _topk_call(x2d, neurons, wqt, bq2d)
    out = _gather_call(topk_idx.reshape(-1), topk_w.reshape(-1), neurons)
    return (
        out.reshape(N_QUERIES, 1, D_MODEL),
        topk_idx.reshape(N_QUERIES, 1, TOP_K),
        topk_w.reshape(N_QUERIES, 1, TOP_K),
    )


# speedup vs baseline: 3.4815x; 1.0062x over previous
import functools

import jax
import jax.numpy as jnp
from jax import lax
from jax.experimental import pallas as pl
from jax.experimental.pallas import tpu as pltpu
from jax.experimental.pallas import tpu_sc as plsc

N_NEURONS = 262144
D_MODEL = 768
TOP_K = 8
N_QUERIES = 8
BLOCK_N = 8192

_NEG_INF = float("-inf")
_BIG_I32 = 2**30


def _insert_sorted(rs, ri, m, j):
    lt = rs < m
    ss = jnp.concatenate([m, rs[:, :-1]], axis=1)
    si = jnp.concatenate([j, ri[:, :-1]], axis=1)
    ltp = ss < m
    new_s = jnp.where(lt, jnp.where(ltp, ss, m), rs)
    new_i = jnp.where(lt, jnp.where(ltp, si, j), ri)
    return new_s, new_i


def _topk_body(x_ref, wqt_ref, bq_ref, n_ref, idx_out, w_out,
               q_s, run_s, run_i, s_buf):
    i = pl.program_id(0)
    nb = pl.num_programs(0)

    @pl.when(i == 0)
    def _init():
        q_s[...] = (
            jnp.dot(x_ref[...], wqt_ref[...], preferred_element_type=jnp.float32)
            + bq_ref[...]
        )
        run_s[...] = jnp.full((N_QUERIES, TOP_K), _NEG_INF, jnp.float32)
        run_i[...] = jnp.full((N_QUERIES, TOP_K), -1, jnp.int32)

    s = jax.lax.dot_general(
        q_s[...], n_ref[...], (((1,), (1,)), ((), ())),
        preferred_element_type=jnp.float32)
    s_buf[...] = s
    col = (jax.lax.broadcasted_iota(jnp.int32, (N_QUERIES, BLOCK_N), 1)
           + i * BLOCK_N)

    thr = run_s[...][:, TOP_K - 1:TOP_K]
    cnt = jnp.sum((s > thr).astype(jnp.int32), axis=1)
    t = jnp.max(cnt)

    for k in range(TOP_K):
        @pl.when(t > k)
        def _step():
            sb = s_buf[...]
            m = jnp.max(sb, axis=1, keepdims=True)
            cand = jnp.where(sb == m, col, _BIG_I32)
            j = jnp.min(cand, axis=1, keepdims=True)
            s_buf[...] = jnp.where(col == j, _NEG_INF, sb)
            ns, ni = _insert_sorted(run_s[...], run_i[...], m, j)
            run_s[...] = ns
            run_i[...] = ni

    @pl.when(i == nb - 1)
    def _fin():
        fs = run_s[...]
        e = jnp.exp(fs - fs[:, :1])
        w_out[...] = e / jnp.sum(e, axis=1, keepdims=True)
        idx_out[...] = run_i[...]


def _topk_call(x2d, neurons, wqt, bq2d):
    nb = N_NEURONS // BLOCK_N
    return pl.pallas_call(
        _topk_body,
        grid=(nb,),
        in_specs=[
            pl.BlockSpec((N_QUERIES, D_MODEL), lambda i: (0, 0)),
            pl.BlockSpec((D_MODEL, D_MODEL), lambda i: (0, 0)),
            pl.BlockSpec((1, D_MODEL), lambda i: (0, 0)),
            pl.BlockSpec((BLOCK_N, D_MODEL), lambda i: (i, 0)),
        ],
        out_specs=[
            pl.BlockSpec((N_QUERIES, TOP_K), lambda i: (0, 0)),
            pl.BlockSpec((N_QUERIES, TOP_K), lambda i: (0, 0)),
        ],
        out_shape=[
            jax.ShapeDtypeStruct((N_QUERIES, TOP_K), jnp.int32),
            jax.ShapeDtypeStruct((N_QUERIES, TOP_K), jnp.float32),
        ],
        scratch_shapes=[
            pltpu.VMEM((N_QUERIES, D_MODEL), jnp.float32),
            pltpu.VMEM((N_QUERIES, TOP_K), jnp.float32),
            pltpu.VMEM((N_QUERIES, TOP_K), jnp.int32),
            pltpu.VMEM((N_QUERIES, BLOCK_N), jnp.float32),
        ],
        compiler_params=pltpu.CompilerParams(
            dimension_semantics=("arbitrary",)),
    )(x2d, wqt, bq2d, neurons)


_LANES = 16


def _sc_gather_body(idx_hbm, wb_hbm, table_hbm, out_hbm,
                    idx_v, wb_v, rows_v, acc_v, sem, sem2):
    wid = lax.axis_index("s")

    @pl.when(wid < N_QUERIES)
    def _():
        b = wid
        cp1 = pltpu.make_async_copy(
            idx_hbm.at[pl.ds(b * TOP_K, TOP_K)], idx_v, sem)
        cp2 = pltpu.make_async_copy(
            wb_hbm.at[pl.ds(b * TOP_K * _LANES, TOP_K * _LANES)], wb_v, sem2)
        cp1.start()
        cp2.start()
        cp1.wait()
        pltpu.async_copy(table_hbm.at[idx_v], rows_v, sem).wait()
        cp2.wait()
        wk = [wb_v[pl.ds(k * _LANES, _LANES)] for k in range(TOP_K)]
        for j in range(D_MODEL // _LANES):
            a = wk[0] * rows_v[0, pl.ds(j * _LANES, _LANES)]
            for k in range(1, TOP_K):
                a = a + wk[k] * rows_v[k, pl.ds(j * _LANES, _LANES)]
            acc_v[pl.ds(j * _LANES, _LANES)] = a
        pltpu.sync_copy(acc_v, out_hbm.at[b])


def _gather_call(idx_flat, w_flat, neurons):
    w_bcast = jnp.broadcast_to(
        w_flat[:, None], (N_QUERIES * TOP_K, _LANES)).reshape(-1)
    mesh = plsc.VectorSubcoreMesh(
        core_axis_name="c", subcore_axis_name="s", num_cores=1)
    f = pl.kernel(
        _sc_gather_body, mesh=mesh,
        out_type=jax.ShapeDtypeStruct((N_QUERIES, D_MODEL), jnp.float32),
        scratch_types=[
            pltpu.VMEM((TOP_K,), jnp.int32),
            pltpu.VMEM((TOP_K * _LANES,), jnp.float32),
            pltpu.VMEM((TOP_K, D_MODEL), jnp.float32),
            pltpu.VMEM((D_MODEL,), jnp.float32),
            pltpu.SemaphoreType.DMA,
            pltpu.SemaphoreType.DMA,
        ],
    )
    return f(idx_flat, w_bcast, neurons)


@jax.jit
def kernel(x, neurons, W_q, b_q):
    x2d = x.reshape(N_QUERIES, D_MODEL)
    wqt = W_q.T
    bq2d = b_q.reshape(1, D_MODEL)
    topk_idx, topk_w = _topk_call(x2d, neurons, wqt, bq2d)
    out = _gather_call(topk_idx.reshape(-1), topk_w.reshape(-1), neurons)
    return (
        out.reshape(N_QUERIES, 1, D_MODEL),
        topk_idx.reshape(N_QUERIES, 1, TOP_K),
        topk_w.reshape(N_QUERIES, 1, TOP_K),
    )


# speedup vs baseline: 3.5014x; 1.0057x over previous
import functools

import jax
import jax.numpy as jnp
from jax import lax
from jax.experimental import pallas as pl
from jax.experimental.pallas import tpu as pltpu
from jax.experimental.pallas import tpu_sc as plsc

N_NEURONS = 262144
D_MODEL = 768
TOP_K = 8
N_QUERIES = 8
BLOCK_N = 8192

_NEG_INF = float("-inf")
_BIG_I32 = 2**30


def _insert_sorted(rs, ri, m, j):
    lt = rs < m
    ss = jnp.concatenate([m, rs[:, :-1]], axis=1)
    si = jnp.concatenate([j, ri[:, :-1]], axis=1)
    ltp = ss < m
    new_s = jnp.where(lt, jnp.where(ltp, ss, m), rs)
    new_i = jnp.where(lt, jnp.where(ltp, si, j), ri)
    return new_s, new_i


def _topk_body(x_ref, wqt_ref, bq_ref, n_ref, idx_out, w_out, wrep_out,
               q_s, run_s, run_i, s_buf):
    i = pl.program_id(0)
    nb = pl.num_programs(0)

    @pl.when(i == 0)
    def _init():
        q_s[...] = (
            jnp.dot(x_ref[...], wqt_ref[...], preferred_element_type=jnp.float32)
            + bq_ref[...]
        )
        run_s[...] = jnp.full((N_QUERIES, TOP_K), _NEG_INF, jnp.float32)
        run_i[...] = jnp.full((N_QUERIES, TOP_K), -1, jnp.int32)

    s = jax.lax.dot_general(
        q_s[...], n_ref[...], (((1,), (1,)), ((), ())),
        preferred_element_type=jnp.float32)
    s_buf[...] = s
    col = (jax.lax.broadcasted_iota(jnp.int32, (N_QUERIES, BLOCK_N), 1)
           + i * BLOCK_N)

    thr = run_s[...][:, TOP_K - 1:TOP_K]
    cnt = jnp.sum((s > thr).astype(jnp.int32), axis=1)
    t = jnp.max(cnt)

    for k in range(TOP_K):
        @pl.when(t > k)
        def _step():
            sb = s_buf[...]
            m = jnp.max(sb, axis=1, keepdims=True)
            cand = jnp.where(sb == m, col, _BIG_I32)
            j = jnp.min(cand, axis=1, keepdims=True)
            s_buf[...] = jnp.where(col == j, _NEG_INF, sb)
            ns, ni = _insert_sorted(run_s[...], run_i[...], m, j)
            run_s[...] = ns
            run_i[...] = ni

    @pl.when(i == nb - 1)
    def _fin():
        fs = run_s[...]
        e = jnp.exp(fs - fs[:, :1])
        w = e / jnp.sum(e, axis=1, keepdims=True)
        w_out[...] = w
        idx_out[...] = run_i[...]
        for c in range(TOP_K):
            wrep_out[:, c * _LANES:(c + 1) * _LANES] = jnp.broadcast_to(
                w[:, c:c + 1], (N_QUERIES, _LANES))


def _topk_call(x2d, neurons, wqt, bq2d):
    nb = N_NEURONS // BLOCK_N
    return pl.pallas_call(
        _topk_body,
        grid=(nb,),
        in_specs=[
            pl.BlockSpec((N_QUERIES, D_MODEL), lambda i: (0, 0)),
            pl.BlockSpec((D_MODEL, D_MODEL), lambda i: (0, 0)),
            pl.BlockSpec((1, D_MODEL), lambda i: (0, 0)),
            pl.BlockSpec((BLOCK_N, D_MODEL), lambda i: (i, 0)),
        ],
        out_specs=[
            pl.BlockSpec((N_QUERIES, TOP_K), lambda i: (0, 0)),
            pl.BlockSpec((N_QUERIES, TOP_K), lambda i: (0, 0)),
            pl.BlockSpec((N_QUERIES, TOP_K * _LANES), lambda i: (0, 0)),
        ],
        out_shape=[
            jax.ShapeDtypeStruct((N_QUERIES, TOP_K), jnp.int32),
            jax.ShapeDtypeStruct((N_QUERIES, TOP_K), jnp.float32),
            jax.ShapeDtypeStruct((N_QUERIES, TOP_K * _LANES), jnp.float32),
        ],
        scratch_shapes=[
            pltpu.VMEM((N_QUERIES, D_MODEL), jnp.float32),
            pltpu.VMEM((N_QUERIES, TOP_K), jnp.float32),
            pltpu.VMEM((N_QUERIES, TOP_K), jnp.int32),
            pltpu.VMEM((N_QUERIES, BLOCK_N), jnp.float32),
        ],
        compiler_params=pltpu.CompilerParams(
            dimension_semantics=("arbitrary",)),
    )(x2d, wqt, bq2d, neurons)


_LANES = 16


def _sc_gather_body(idx_hbm, wb_hbm, table_hbm, out_hbm,
                    idx_v, wb_v, rows_v, acc_v, sem, sem2):
    wid = lax.axis_index("s")

    @pl.when(wid < N_QUERIES)
    def _():
        b = wid
        cp1 = pltpu.make_async_copy(
            idx_hbm.at[pl.ds(b * TOP_K, TOP_K)], idx_v, sem)
        cp2 = pltpu.make_async_copy(
            wb_hbm.at[pl.ds(b * TOP_K * _LANES, TOP_K * _LANES)], wb_v, sem2)
        cp1.start()
        cp2.start()
        cp1.wait()
        pltpu.async_copy(table_hbm.at[idx_v], rows_v, sem).wait()
        cp2.wait()
        wk = [wb_v[pl.ds(k * _LANES, _LANES)] for k in range(TOP_K)]
        for j in range(D_MODEL // _LANES):
            a = wk[0] * rows_v[0, pl.ds(j * _LANES, _LANES)]
            for k in range(1, TOP_K):
                a = a + wk[k] * rows_v[k, pl.ds(j * _LANES, _LANES)]
            acc_v[pl.ds(j * _LANES, _LANES)] = a
        pltpu.sync_copy(acc_v, out_hbm.at[b])


def _gather_call(idx_flat, w_bcast, neurons):
    mesh = plsc.VectorSubcoreMesh(
        core_axis_name="c", subcore_axis_name="s", num_cores=1)
    f = pl.kernel(
        _sc_gather_body, mesh=mesh,
        out_type=jax.ShapeDtypeStruct((N_QUERIES, D_MODEL), jnp.float32),
        scratch_types=[
            pltpu.VMEM((TOP_K,), jnp.int32),
            pltpu.VMEM((TOP_K * _LANES,), jnp.float32),
            pltpu.VMEM((TOP_K, D_MODEL), jnp.float32),
            pltpu.VMEM((D_MODEL,), jnp.float32),
            pltpu.SemaphoreType.DMA,
            pltpu.SemaphoreType.DMA,
        ],
    )
    return f(idx_flat, w_bcast, neurons)


@jax.jit
def kernel(x, neurons, W_q, b_q):
    x2d = x.reshape(N_QUERIES, D_MODEL)
    wqt = W_q.T
    bq2d = b_q.reshape(1, D_MODEL)
    topk_idx, topk_w, topk_wrep = _topk_call(x2d, neurons, wqt, bq2d)
    out = _gather_call(topk_idx.reshape(-1), topk_wrep.reshape(-1), neurons)
    return (
        out.reshape(N_QUERIES, 1, D_MODEL),
        topk_idx.reshape(N_QUERIES, 1, TOP_K),
        topk_w.reshape(N_QUERIES, 1, TOP_K),
    )
